# baseline (speedup 1.0000x reference)
import functools

import jax
import jax.numpy as jnp
from jax import lax
from jax.experimental import pallas as pl
from jax.experimental.pallas import tpu as pltpu
from jax.experimental.pallas import tpu_sc as plsc

N, E, D, H = 10000, 320000, 128, 512
NP = 10240
BR = 1024
NSUB = 128
NSEG = 80
CH = 32
EPAD = 334336
HC = H // 16


def _proj_body(x_ref, w_ref, ol_ref, or_ref):
    r = jnp.dot(x_ref[...], w_ref[...], preferred_element_type=jnp.float32)
    ol_ref[...] = r[:, :H]
    or_ref[...] = r[:, H:]


def _proj(x, w):
    K = w.shape[0]
    return pl.pallas_call(
        _proj_body,
        grid=(NP // BR,),
        in_specs=[
            pl.BlockSpec((BR, K), lambda i: (i, 0)),
            pl.BlockSpec((K, 2 * H), lambda i: (0, 0)),
        ],
        out_specs=[pl.BlockSpec((BR, H), lambda i: (i, 0)),
                   pl.BlockSpec((BR, H), lambda i: (i, 0))],
        out_shape=[jax.ShapeDtypeStruct((NP, H), jnp.float32),
                   jax.ShapeDtypeStruct((NP, H), jnp.float32)],
    )(x, w)


def _act_proj_body(g_ref, b_ref, w_ref, ol_ref, or_ref):
    h = jnp.tanh(g_ref[...] + b_ref[...])
    r = jnp.dot(h, w_ref[...], preferred_element_type=jnp.float32)
    ol_ref[...] = r[:, :H]
    or_ref[...] = r[:, H:]


def _act_proj(g, b, w):
    K = w.shape[0]
    return pl.pallas_call(
        _act_proj_body,
        grid=(NP // BR,),
        in_specs=[
            pl.BlockSpec((BR, K), lambda i: (i, 0)),
            pl.BlockSpec((1, K), lambda i: (0, 0)),
            pl.BlockSpec((K, 2 * H), lambda i: (0, 0)),
        ],
        out_specs=[pl.BlockSpec((BR, H), lambda i: (i, 0)),
                   pl.BlockSpec((BR, H), lambda i: (i, 0))],
        out_shape=[jax.ShapeDtypeStruct((NP, H), jnp.float32),
                   jax.ShapeDtypeStruct((NP, H), jnp.float32)],
    )(g, b, w)


def _head_body(g_ref, b3_ref, w1_ref, b1_ref, w2_ref, b2_ref, w3_ref, b3p_ref, o_ref):
    h = jnp.tanh(g_ref[...] + b3_ref[...])
    h = h @ w1_ref[...] + b1_ref[...]
    h = jnp.where(h > 0, h, 0.01 * h)
    h = h @ w2_ref[...] + b2_ref[...]
    h = jnp.where(h > 0, h, 0.01 * h)
    o_ref[...] = h @ w3_ref[...] + b3p_ref[...]


def _head(g, b3, w1, b1, w2, b2, w3p, b3p):
    return pl.pallas_call(
        _head_body,
        grid=(NP // BR,),
        in_specs=[
            pl.BlockSpec((BR, H), lambda i: (i, 0)),
            pl.BlockSpec((1, H), lambda i: (0, 0)),
            pl.BlockSpec((H, H), lambda i: (0, 0)),
            pl.BlockSpec((1, H), lambda i: (0, 0)),
            pl.BlockSpec((H, H), lambda i: (0, 0)),
            pl.BlockSpec((1, H), lambda i: (0, 0)),
            pl.BlockSpec((H, 128), lambda i: (0, 0)),
            pl.BlockSpec((1, 128), lambda i: (0, 0)),
        ],
        out_specs=pl.BlockSpec((BR, 128), lambda i: (i, 0)),
        out_shape=jax.ShapeDtypeStruct((NP, 128), jnp.float32),
    )(g, b3, w1, b1, w2, b2, w3p, b3p)


def _edge_body(xl_hbm, xr_hbm, att_hbm, src_hbm, dst_hbm, eoff_hbm,
               out_hbm, xrs, slab, rows_a, rows_b, srcv_a, srcv_b,
               dstv_a, dstv_b, attv, maxt, dent, eoffv, sem_a, sem_b):
    cid = lax.axis_index("c")
    sid = lax.axis_index("s")
    wid = sid * 2 + cid
    lane = lax.iota(jnp.int32, 16)
    m0 = lane == 0

    pltpu.sync_copy(att_hbm, attv)
    pltpu.sync_copy(eoff_hbm, eoffv)

    def _sub(r, _carry):
        k = wid * 4 + r
        lo = k * NSEG
        nseg = jnp.minimum(NSEG, N - lo)
        hi = lo + nseg
        ev = eoffv[pl.ds(k, 16)]
        e_s = pl.multiple_of(ev[0], CH)
        e_e = ev[1]
        nch = (e_e - e_s) // CH

        pltpu.sync_copy(xr_hbm.at[pl.ds(lo, NSEG)], xrs)

        def _zrow(rr, _):
            row = rr // 4
            cb = (rr % 4) * 128
            for u in range(8):
                slab[row, pl.ds(cb + u * 16, 16)] = jnp.zeros((16,), jnp.float32)
            return 0
        lax.fori_loop(0, NSEG * 4, _zrow, 0)

        def _init_tab(t, _):
            maxt[pl.ds(t * 16, 16)] = jnp.full((16,), -3e38, jnp.float32)
            dent[pl.ds(t * 16, 16)] = jnp.zeros((16,), jnp.float32)
            return 0
        lax.fori_loop(0, NSEG // 16, _init_tab, 0)

        def _load_sd(ci, sv, dv):
            base = pl.multiple_of(e_s + ci * CH, CH)
            pltpu.sync_copy(src_hbm.at[pl.ds(base, CH)], sv)
            pltpu.sync_copy(dst_hbm.at[pl.ds(base, CH + 16)], dv)

        def _process(rows, dstv):
            def _edge(i, _):
                d = dstv[pl.ds(i, 16)][0]
                seg = jnp.clip(d - lo, 0, nseg - 1)

                def _hc8(h8, acc):
                    for u in range(8):
                        o = h8 * 128 + u * 16
                        v = rows[i, pl.ds(o, 16)] + xrs[seg, pl.ds(o, 16)]
                        v = jnp.maximum(v, 0.2 * v)
                        acc = acc + v * attv[pl.ds(o, 16)]
                    return acc
                acc = lax.fori_loop(0, HC // 8, _hc8, jnp.zeros((16,), jnp.float32))
                l = jnp.sum(acc)

                @pl.when(d < hi)
                def _():
                    segv = jnp.full((16,), seg, jnp.int32)
                    mo = plsc.load_gather(maxt, [segv])
                    do = plsc.load_gather(dent, [segv])
                    lv = jnp.full((16,), l, jnp.float32)
                    mn = jnp.maximum(mo, lv)
                    f = jnp.exp(mo - mn)
                    p = jnp.exp(lv - mn)
                    plsc.store_scatter(maxt, [segv], mn, mask=m0)
                    plsc.store_scatter(dent, [segv], do * f + p, mask=m0)

                    def _hca(h8, _):
                        for u in range(8):
                            o = h8 * 128 + u * 16
                            slab[seg, pl.ds(o, 16)] = (
                                slab[seg, pl.ds(o, 16)] * f
                                + p * rows[i, pl.ds(o, 16)])
                        return 0
                    lax.fori_loop(0, HC // 8, _hca, 0)
                return 0
            lax.fori_loop(0, CH, _edge, 0)

        _load_sd(0, srcv_a, dstv_a)
        pltpu.async_copy(xl_hbm.at[srcv_a], rows_a, sem_a)

        def _pair(j, _):
            c = j * 2
            _load_sd(c + 1, srcv_b, dstv_b)
            cpb = pltpu.async_copy(xl_hbm.at[srcv_b], rows_b, sem_b)
            pltpu.make_async_copy(xl_hbm.at[srcv_a], rows_a, sem_a).wait()
            _process(rows_a, dstv_a)
            _load_sd(c + 2, srcv_a, dstv_a)
            pltpu.async_copy(xl_hbm.at[srcv_a], rows_a, sem_a)
            cpb.wait()
            _process(rows_b, dstv_b)
            return 0
        lax.fori_loop(0, nch // 2, _pair, 0)

        pltpu.make_async_copy(xl_hbm.at[srcv_a], rows_a, sem_a).wait()

        @pl.when(nch % 2 == 1)
        def _():
            _process(rows_a, dstv_a)

        def _nrow(rr, _):
            rv = jnp.full((16,), rr, jnp.int32)
            dv = plsc.load_gather(dent, [rv])
            inv = 1.0 / jnp.maximum(dv, 1e-30)

            def _hcn(h8, _):
                for u in range(8):
                    o = h8 * 128 + u * 16
                    slab[rr, pl.ds(o, 16)] = slab[rr, pl.ds(o, 16)] * inv
                return 0
            lax.fori_loop(0, HC // 8, _hcn, 0)
            return 0
        lax.fori_loop(0, NSEG, _nrow, 0)

        pltpu.sync_copy(slab, out_hbm.at[pl.ds(lo, NSEG)])
        return 0
    lax.fori_loop(0, 4, _sub, 0)


def _edge_stage_sc(xl, xr, att, srcp, dstp, eoff):
    mesh = plsc.VectorSubcoreMesh(core_axis_name="c", subcore_axis_name="s")
    f = pl.kernel(
        _edge_body,
        out_type=jax.ShapeDtypeStruct((NP, H), jnp.float32),
        mesh=mesh,
        compiler_params=pltpu.CompilerParams(needs_layout_passes=False),
        scratch_types=[
            pltpu.VMEM((NSEG, H), jnp.float32),
            pltpu.VMEM((NSEG, H), jnp.float32),
            pltpu.VMEM((CH, H), jnp.float32),
            pltpu.VMEM((CH, H), jnp.float32),
            pltpu.VMEM((CH,), jnp.int32),
            pltpu.VMEM((CH,), jnp.int32),
            pltpu.VMEM((CH + 16,), jnp.int32),
            pltpu.VMEM((CH + 16,), jnp.int32),
            pltpu.VMEM((H,), jnp.float32),
            pltpu.VMEM((NSEG,), jnp.float32),
            pltpu.VMEM((NSEG,), jnp.float32),
            pltpu.VMEM((160,), jnp.int32),
            pltpu.SemaphoreType.DMA,
            pltpu.SemaphoreType.DMA,
        ],
    )
    return f(xl, xr, att, srcp, dstp, eoff)


def _prep_edges(edge_index):
    loops = jnp.arange(N, dtype=edge_index.dtype)
    src = jnp.concatenate([edge_index[0], loops])
    dst = jnp.concatenate([edge_index[1], loops])
    order = jnp.argsort(dst)
    src_s = src[order].astype(jnp.int32)
    dst_s = dst[order].astype(jnp.int32)
    bounds = (jnp.arange(NSUB + 1, dtype=jnp.int32) * NSEG).astype(dst_s.dtype)
    seg_start = jnp.searchsorted(dst_s, bounds[:-1], side="left").astype(jnp.int32)
    seg_end = jnp.searchsorted(dst_s, bounds[1:], side="left").astype(jnp.int32)
    cnt = seg_end - seg_start
    pcnt = ((cnt + CH - 1) // CH) * CH
    off = jnp.concatenate([jnp.zeros((1,), jnp.int32), jnp.cumsum(pcnt)]).astype(jnp.int32)
    parange = jnp.arange(EPAD, dtype=jnp.int32)
    b = jnp.clip(jnp.searchsorted(off, parange, side="right").astype(jnp.int32) - 1,
                 0, NSUB - 1)
    rel = parange - off[b]
    valid = rel < cnt[b]
    idx = jnp.clip(seg_start[b] + jnp.minimum(rel, jnp.maximum(cnt[b] - 1, 0)),
                   0, E + N - 1)
    srcp = jnp.where(valid, src_s[idx], 0)
    dstp = jnp.where(valid, dst_s[idx], N)
    eoff = jnp.concatenate([off, jnp.full((160 - NSUB - 1,), off[NSUB], jnp.int32)])
    return srcp, dstp, eoff


def kernel(x, edge_index, Wl1, Wr1, att1, b1, Wl2, Wr2, att2, b2,
           Wl3, Wr3, att3, b3, lW1, lb1, lW2, lb2, lW3, lb3):
    srcp, dstp, eoff = _prep_edges(edge_index)

    xp = jnp.pad(x, ((0, NP - N), (0, 0)))
    w1 = jnp.concatenate([Wl1, Wr1], axis=1)
    w2 = jnp.concatenate([Wl2, Wr2], axis=1)
    w3 = jnp.concatenate([Wl3, Wr3], axis=1)
    w3p = jnp.pad(lW3, ((0, 0), (0, 127)))
    b3p = jnp.pad(lb3, (0, 127))

    xl, xr = _proj(xp, w1)
    g1 = _edge_stage_sc(xl, xr, att1, srcp, dstp, eoff)
    xl, xr = _act_proj(g1, b1[None, :], w2)
    g2 = _edge_stage_sc(xl, xr, att2, srcp, dstp, eoff)
    xl, xr = _act_proj(g2, b2[None, :], w3)
    g3 = _edge_stage_sc(xl, xr, att3, srcp, dstp, eoff)
    y = _head(g3, b3[None, :], lW1, lb1[None, :], lW2, lb2[None, :],
              w3p, b3p[None, :])
    return y[:N, :1]


# speedup vs baseline: 1.8438x; 1.8438x over previous
import functools

import jax
import jax.numpy as jnp
from jax import lax
from jax.experimental import pallas as pl
from jax.experimental.pallas import tpu as pltpu
from jax.experimental.pallas import tpu_sc as plsc

N, E, D, H = 10000, 320000, 128, 512
NP = 10240
BR = 1024
NSUB = 128
NSEG = 80
CH = 32
EPAD = 334336
HC = H // 16


def _proj_body(x_ref, w_ref, ol_ref, or_ref):
    r = jnp.dot(x_ref[...], w_ref[...], preferred_element_type=jnp.float32)
    ol_ref[...] = r[:, :H]
    or_ref[...] = r[:, H:]


def _proj(x, w):
    K = w.shape[0]
    return pl.pallas_call(
        _proj_body,
        grid=(NP // BR,),
        in_specs=[
            pl.BlockSpec((BR, K), lambda i: (i, 0)),
            pl.BlockSpec((K, 2 * H), lambda i: (0, 0)),
        ],
        out_specs=[pl.BlockSpec((BR, H), lambda i: (i, 0)),
                   pl.BlockSpec((BR, H), lambda i: (i, 0))],
        out_shape=[jax.ShapeDtypeStruct((NP, H), jnp.float32),
                   jax.ShapeDtypeStruct((NP, H), jnp.float32)],
    )(x, w)


def _act_proj_body(g_ref, b_ref, w_ref, ol_ref, or_ref):
    h = jnp.tanh(g_ref[...] + b_ref[...])
    r = jnp.dot(h, w_ref[...], preferred_element_type=jnp.float32)
    ol_ref[...] = r[:, :H]
    or_ref[...] = r[:, H:]


def _act_proj(g, b, w):
    K = w.shape[0]
    return pl.pallas_call(
        _act_proj_body,
        grid=(NP // BR,),
        in_specs=[
            pl.BlockSpec((BR, K), lambda i: (i, 0)),
            pl.BlockSpec((1, K), lambda i: (0, 0)),
            pl.BlockSpec((K, 2 * H), lambda i: (0, 0)),
        ],
        out_specs=[pl.BlockSpec((BR, H), lambda i: (i, 0)),
                   pl.BlockSpec((BR, H), lambda i: (i, 0))],
        out_shape=[jax.ShapeDtypeStruct((NP, H), jnp.float32),
                   jax.ShapeDtypeStruct((NP, H), jnp.float32)],
    )(g, b, w)


def _head_body(g_ref, b3_ref, w1_ref, b1_ref, w2_ref, b2_ref, w3_ref, b3p_ref, o_ref):
    h = jnp.tanh(g_ref[...] + b3_ref[...])
    h = h @ w1_ref[...] + b1_ref[...]
    h = jnp.where(h > 0, h, 0.01 * h)
    h = h @ w2_ref[...] + b2_ref[...]
    h = jnp.where(h > 0, h, 0.01 * h)
    o_ref[...] = h @ w3_ref[...] + b3p_ref[...]


def _head(g, b3, w1, b1, w2, b2, w3p, b3p):
    return pl.pallas_call(
        _head_body,
        grid=(NP // BR,),
        in_specs=[
            pl.BlockSpec((BR, H), lambda i: (i, 0)),
            pl.BlockSpec((1, H), lambda i: (0, 0)),
            pl.BlockSpec((H, H), lambda i: (0, 0)),
            pl.BlockSpec((1, H), lambda i: (0, 0)),
            pl.BlockSpec((H, H), lambda i: (0, 0)),
            pl.BlockSpec((1, H), lambda i: (0, 0)),
            pl.BlockSpec((H, 128), lambda i: (0, 0)),
            pl.BlockSpec((1, 128), lambda i: (0, 0)),
        ],
        out_specs=pl.BlockSpec((BR, 128), lambda i: (i, 0)),
        out_shape=jax.ShapeDtypeStruct((NP, 128), jnp.float32),
    )(g, b3, w1, b1, w2, b2, w3p, b3p)


def _edge_body(xl_hbm, xr_hbm, att_hbm, src_hbm, dst_hbm, eoff_hbm,
               out_hbm, xrs, slab, rows_a, rows_b, srcv_a, srcv_b,
               dstv_a, dstv_b, attv, maxt, dent, eoffv, sem_a, sem_b):
    cid = lax.axis_index("c")
    sid = lax.axis_index("s")
    wid = sid * 2 + cid
    lane = lax.iota(jnp.int32, 16)
    m0 = lane == 0

    pltpu.sync_copy(att_hbm, attv)
    pltpu.sync_copy(eoff_hbm, eoffv)

    def _sub(r, _carry):
        k = wid * 4 + r
        lo = k * NSEG
        nseg = jnp.minimum(NSEG, N - lo)
        hi = lo + nseg
        ev = eoffv[pl.ds(k, 16)]
        e_s = pl.multiple_of(ev[0], CH)
        e_e = ev[1]
        nch = (e_e - e_s) // CH

        pltpu.sync_copy(xr_hbm.at[pl.ds(lo, NSEG)], xrs)

        def _zrow(rr, _):
            row = rr // 4
            cb = (rr % 4) * 128
            for u in range(8):
                slab[row, pl.ds(cb + u * 16, 16)] = jnp.zeros((16,), jnp.float32)
            return 0
        lax.fori_loop(0, NSEG * 4, _zrow, 0)

        def _init_tab(t, _):
            maxt[pl.ds(t * 16, 16)] = jnp.full((16,), -3e38, jnp.float32)
            dent[pl.ds(t * 16, 16)] = jnp.zeros((16,), jnp.float32)
            return 0
        lax.fori_loop(0, NSEG // 16, _init_tab, 0)

        def _load_sd(ci, sv, dv):
            base = pl.multiple_of(e_s + ci * CH, CH)
            pltpu.sync_copy(src_hbm.at[pl.ds(base, CH)], sv)
            pltpu.sync_copy(dst_hbm.at[pl.ds(base, CH + 16)], dv)

        def _process(rows, dstv):
            def _edge(i, _):
                d = dstv[pl.ds(i, 16)][0]
                seg = jnp.clip(d - lo, 0, nseg - 1)

                def _hc8(h8, acc):
                    for u in range(8):
                        o = h8 * 128 + u * 16
                        v = rows[i, pl.ds(o, 16)] + xrs[seg, pl.ds(o, 16)]
                        v = jnp.maximum(v, 0.2 * v)
                        acc = acc + v * attv[pl.ds(o, 16)]
                    return acc
                acc = lax.fori_loop(0, HC // 8, _hc8, jnp.zeros((16,), jnp.float32))
                l = jnp.sum(acc)

                @pl.when(d < hi)
                def _():
                    segv = jnp.full((16,), seg, jnp.int32)
                    mo = plsc.load_gather(maxt, [segv])
                    do = plsc.load_gather(dent, [segv])
                    lv = jnp.full((16,), l, jnp.float32)
                    mn = jnp.maximum(mo, lv)
                    f = jnp.exp(mo - mn)
                    p = jnp.exp(lv - mn)
                    plsc.store_scatter(maxt, [segv], mn, mask=m0)
                    plsc.store_scatter(dent, [segv], do * f + p, mask=m0)

                    def _hca(h8, _):
                        for u in range(8):
                            o = h8 * 128 + u * 16
                            slab[seg, pl.ds(o, 16)] = (
                                slab[seg, pl.ds(o, 16)] * f
                                + p * rows[i, pl.ds(o, 16)])
                        return 0
                    lax.fori_loop(0, HC // 8, _hca, 0)
                return 0
            lax.fori_loop(0, CH, _edge, 0)

        _load_sd(0, srcv_a, dstv_a)
        pltpu.async_copy(xl_hbm.at[srcv_a], rows_a, sem_a)

        def _pair(j, _):
            c = j * 2
            _load_sd(c + 1, srcv_b, dstv_b)
            cpb = pltpu.async_copy(xl_hbm.at[srcv_b], rows_b, sem_b)
            pltpu.make_async_copy(xl_hbm.at[srcv_a], rows_a, sem_a).wait()
            _process(rows_a, dstv_a)
            _load_sd(c + 2, srcv_a, dstv_a)
            pltpu.async_copy(xl_hbm.at[srcv_a], rows_a, sem_a)
            cpb.wait()
            _process(rows_b, dstv_b)
            return 0
        lax.fori_loop(0, nch // 2, _pair, 0)

        pltpu.make_async_copy(xl_hbm.at[srcv_a], rows_a, sem_a).wait()

        @pl.when(nch % 2 == 1)
        def _():
            _process(rows_a, dstv_a)

        def _nrow(rr, _):
            rv = jnp.full((16,), rr, jnp.int32)
            dv = plsc.load_gather(dent, [rv])
            inv = 1.0 / jnp.maximum(dv, 1e-30)

            def _hcn(h8, _):
                for u in range(8):
                    o = h8 * 128 + u * 16
                    slab[rr, pl.ds(o, 16)] = slab[rr, pl.ds(o, 16)] * inv
                return 0
            lax.fori_loop(0, HC // 8, _hcn, 0)
            return 0
        lax.fori_loop(0, NSEG, _nrow, 0)

        pltpu.sync_copy(slab, out_hbm.at[pl.ds(lo, NSEG)])
        return 0
    lax.fori_loop(0, 4, _sub, 0)


def _edge_stage_sc(xl, xr, att, srcp, dstp, eoff):
    mesh = plsc.VectorSubcoreMesh(core_axis_name="c", subcore_axis_name="s")
    f = pl.kernel(
        _edge_body,
        out_type=jax.ShapeDtypeStruct((NP, H), jnp.float32),
        mesh=mesh,
        compiler_params=pltpu.CompilerParams(needs_layout_passes=False),
        scratch_types=[
            pltpu.VMEM((NSEG, H), jnp.float32),
            pltpu.VMEM((NSEG, H), jnp.float32),
            pltpu.VMEM((CH, H), jnp.float32),
            pltpu.VMEM((CH, H), jnp.float32),
            pltpu.VMEM((CH,), jnp.int32),
            pltpu.VMEM((CH,), jnp.int32),
            pltpu.VMEM((CH + 16,), jnp.int32),
            pltpu.VMEM((CH + 16,), jnp.int32),
            pltpu.VMEM((H,), jnp.float32),
            pltpu.VMEM((NSEG,), jnp.float32),
            pltpu.VMEM((NSEG,), jnp.float32),
            pltpu.VMEM((160,), jnp.int32),
            pltpu.SemaphoreType.DMA,
            pltpu.SemaphoreType.DMA,
        ],
    )
    return f(xl, xr, att, srcp, dstp, eoff)


def _prep_edges(edge_index):
    loops = jnp.arange(N, dtype=edge_index.dtype)
    src = jnp.concatenate([edge_index[0], loops])
    dst = jnp.concatenate([edge_index[1], loops])
    order = jnp.argsort(dst)
    src_s = src[order].astype(jnp.int32)
    dst_s = dst[order].astype(jnp.int32)
    bounds = (jnp.arange(NSUB + 1, dtype=jnp.int32) * NSEG).astype(dst_s.dtype)
    seg_start = jnp.searchsorted(dst_s, bounds[:-1], side="left").astype(jnp.int32)
    seg_end = jnp.searchsorted(dst_s, bounds[1:], side="left").astype(jnp.int32)
    cnt = seg_end - seg_start
    pcnt = ((cnt + CH - 1) // CH) * CH
    off = jnp.concatenate([jnp.zeros((1,), jnp.int32), jnp.cumsum(pcnt)]).astype(jnp.int32)
    parange = jnp.arange(EPAD, dtype=jnp.int32)
    b = jnp.clip(jnp.sum((parange[:, None] >= off[None, 1:NSUB + 1]).astype(jnp.int32),
                         axis=1), 0, NSUB - 1)
    rel = parange - off[b]
    valid = rel < cnt[b]
    idx = jnp.clip(seg_start[b] + jnp.minimum(rel, jnp.maximum(cnt[b] - 1, 0)),
                   0, E + N - 1)
    srcp = jnp.where(valid, src_s[idx], 0)
    dstp = jnp.where(valid, dst_s[idx], N)
    eoff = jnp.concatenate([off, jnp.full((160 - NSUB - 1,), off[NSUB], jnp.int32)])
    return srcp, dstp, eoff


def kernel(x, edge_index, Wl1, Wr1, att1, b1, Wl2, Wr2, att2, b2,
           Wl3, Wr3, att3, b3, lW1, lb1, lW2, lb2, lW3, lb3):
    srcp, dstp, eoff = _prep_edges(edge_index)

    xp = jnp.pad(x, ((0, NP - N), (0, 0)))
    w1 = jnp.concatenate([Wl1, Wr1], axis=1)
    w2 = jnp.concatenate([Wl2, Wr2], axis=1)
    w3 = jnp.concatenate([Wl3, Wr3], axis=1)
    w3p = jnp.pad(lW3, ((0, 0), (0, 127)))
    b3p = jnp.pad(lb3, (0, 127))

    xl, xr = _proj(xp, w1)
    g1 = _edge_stage_sc(xl, xr, att1, srcp, dstp, eoff)
    xl, xr = _act_proj(g1, b1[None, :], w2)
    g2 = _edge_stage_sc(xl, xr, att2, srcp, dstp, eoff)
    xl, xr = _act_proj(g2, b2[None, :], w3)
    g3 = _edge_stage_sc(xl, xr, att3, srcp, dstp, eoff)
    y = _head(g3, b3[None, :], lW1, lb1[None, :], lW2, lb2[None, :],
              w3p, b3p[None, :])
    return y[:N, :1]
